# baseline (speedup 1.0000x reference)
import jax
import jax.numpy as jnp
from jax import lax
from jax.experimental import pallas as pl
from jax.experimental.pallas import tpu as pltpu
from jax.experimental.pallas import tpu_sc as plsc

N = 10000
NP_ = 10240
E = 160000
D = 256
K = 10
COLS = 64
NSUB = 16
EPT = 10240
BB = 128
NB = EPT // BB
NPT = NP_ // NSUB
RC = 64
NRC = NPT // RC
RV = COLS // 16
RING = 4


def _rsqrt16(d):
    y = jnp.full((16,), 64.0, jnp.float32)
    for _ in range(12):
        y = jnp.float32(0.5) * (y + d / y)
    v = jnp.float32(1.0) / y
    return jnp.where(d == jnp.float32(0.0), jnp.float32(jnp.inf), v)


def _gpr_body(feats, src2, dst_rs, temp32, out,
              s_tab, y_acc,
              src_v, dst_v, rows, fin, zbuf, dchunk, temp_v,
              gsem, ssem, isem, osem):
    c = lax.axis_index("c")
    sid = lax.axis_index("s")
    nbase = sid * NPT

    zero = jnp.zeros((16,), jnp.float32)
    one = jnp.ones((16,), jnp.float32)

    def _zrow(i, carry):
        for u in range(RV):
            zbuf[i, pl.ds(16 * u, 16)] = zero
        return carry
    lax.fori_loop(0, RC, _zrow, 0)

    def _orow(i, carry):
        for u in range(RV):
            rows[0, i, pl.ds(16 * u, 16)] = one
        return carry
    lax.fori_loop(0, BB, _orow, 0)

    pltpu.sync_copy(temp32, temp_v)
    pltpu.sync_copy(src2.at[c, sid], src_v)
    pltpu.sync_copy(dst_rs.at[sid], dst_v)

    def _zero_y():
        for g in range(0, NRC, 5):
            for z in range(g, g + 5):
                pltpu.async_copy(zbuf, y_acc.at[pl.ds(nbase + RC * z, RC)],
                                 osem)
            for z in range(g, g + 5):
                pltpu.make_async_copy(
                    zbuf, y_acc.at[pl.ds(nbase + RC * z, RC)], osem).wait()

    _zero_y()
    plsc.subcore_barrier()
    def _degj(j, carry):
        pltpu.async_copy(rows.at[0], y_acc.at[dst_v.at[j]], ssem, add=True)

        @pl.when(j >= 8)
        def _():
            pltpu.make_async_copy(rows.at[0], y_acc.at[dst_v.at[j]],
                                  ssem).wait()
        return carry
    lax.fori_loop(0, NB, _degj, 0)
    for _ in range(8):
        pltpu.make_async_copy(rows.at[0], y_acc.at[dst_v.at[0]], ssem).wait()
    plsc.subcore_barrier()

    for q in range(NRC):
        pltpu.sync_copy(y_acc.at[pl.ds(nbase + RC * q, RC)],
                        fin.at[0, pl.ds(0, RC)])
        def _nrm(i, carry):
            dchunk[RC * q + i] = _rsqrt16(fin[0, i, pl.ds(0, 16)])
            return carry
        lax.fori_loop(0, RC, _nrm, 0)
    _zero_y()

    t0 = temp_v[pl.ds(0, 16)][0]
    for h in range(2):
        boff = (2 * c + h) * NP_
        cbase = c * 128 + h * COLS
        pltpu.async_copy(feats.at[pl.ds(nbase, RC), pl.ds(cbase, COLS)],
                         fin.at[0, pl.ds(0, RC)], isem)

        @pl.loop(0, NRC, step=2)
        def _init(q0):
            for b in range(2):
                q = q0 + b
                rbase = nbase + RC * q
                pltpu.make_async_copy(
                    feats.at[pl.ds(rbase, RC), pl.ds(cbase, COLS)],
                    fin.at[b, pl.ds(0, RC)], isem).wait()

                def _irow(i, carry):
                    nrm = dchunk[RC * q + i]
                    for u in range(RV):
                        x = fin[b, i, pl.ds(16 * u, 16)]
                        fin[b, RC + i, pl.ds(16 * u, 16)] = x * nrm
                        fin[b, i, pl.ds(16 * u, 16)] = x * t0
                    return carry
                lax.fori_loop(0, RC, _irow, 0)
                pltpu.async_copy(fin.at[b, pl.ds(RC, RC)],
                                 s_tab.at[pl.ds(boff + rbase, RC)], osem)
                pltpu.async_copy(
                    fin.at[b, pl.ds(0, RC)],
                    out.at[pl.ds(rbase, RC), pl.ds(cbase, COLS)], osem)

                @pl.when(q >= 1)
                def _():
                    rb1 = nbase + RC * (q - 1)
                    pltpu.make_async_copy(
                        fin.at[1 - b, pl.ds(RC, RC)],
                        s_tab.at[pl.ds(boff + rb1, RC)], osem).wait()
                    pltpu.make_async_copy(
                        fin.at[1 - b, pl.ds(0, RC)],
                        out.at[pl.ds(rb1, RC), pl.ds(cbase, COLS)],
                        osem).wait()

                @pl.when(q + 1 < NRC)
                def _():
                    rb2 = nbase + RC * (q + 1)
                    pltpu.async_copy(
                        feats.at[pl.ds(rb2, RC), pl.ds(cbase, COLS)],
                        fin.at[1 - b, pl.ds(0, RC)], isem)

        rbl = nbase + RC * (NRC - 1)
        pltpu.make_async_copy(fin.at[1, pl.ds(RC, RC)],
                              s_tab.at[pl.ds(boff + rbl, RC)], osem).wait()
        pltpu.make_async_copy(fin.at[1, pl.ds(0, RC)],
                              out.at[pl.ds(rbl, RC), pl.ds(cbase, COLS)],
                              osem).wait()

    def _hop(k, carry):
        gamma = temp_v[pl.ds(k + 1, 16)][0]
        for h in range(2):
            boff = (2 * c + h) * NP_
            plsc.subcore_barrier()
            pltpu.async_copy(s_tab.at[src_v.at[0]], rows.at[0], gsem)

            @pl.loop(0, NB, step=2)
            def _edge(j0):
                for b in range(2):
                    jj = j0 + b
                    pltpu.make_async_copy(s_tab.at[src_v.at[jj]],
                                          rows.at[b], gsem).wait()

                    @pl.when(jj >= 1)
                    def _():
                        pltpu.make_async_copy(
                            rows.at[1 - b],
                            y_acc.at[dst_v.at[jj]], ssem).wait()

                    @pl.when(jj + 1 < NB)
                    def _():
                        pltpu.async_copy(s_tab.at[src_v.at[jj + 1]],
                                         rows.at[1 - b], gsem)

                    pltpu.async_copy(rows.at[b], y_acc.at[dst_v.at[jj]],
                                     ssem, add=True)

            pltpu.make_async_copy(rows.at[1], y_acc.at[dst_v.at[0]],
                                  ssem).wait()

            step = NP_ if h == 0 else -NP_
            def _tog(j, carry2):
                for u in range(BB // 16):
                    src_v[j, pl.ds(16 * u, 16)] = (
                        src_v[j, pl.ds(16 * u, 16)] + jnp.int32(step))
                return carry2
            lax.fori_loop(0, NB, _tog, 0)

            plsc.subcore_barrier()
            cbase = c * 128 + h * COLS

            @pl.loop(0, NRC)
            def _fint(q):
                rbase = nbase + RC * q
                pltpu.sync_copy(y_acc.at[pl.ds(rbase, RC)],
                                fin.at[0, pl.ds(0, RC)])
                pltpu.sync_copy(zbuf, y_acc.at[pl.ds(rbase, RC)])
                pltpu.sync_copy(
                    out.at[pl.ds(rbase, RC), pl.ds(cbase, COLS)],
                    fin.at[0, pl.ds(RC, RC)])

                def _frow(i, fcarry):
                    nrm = dchunk[RC * q + i]
                    for u in range(RV):
                        y = fin[0, i, pl.ds(16 * u, 16)]
                        xp = y * nrm
                        fin[0, RC + i, pl.ds(16 * u, 16)] = (
                            fin[0, RC + i, pl.ds(16 * u, 16)]
                            + gamma * xp)
                        fin[0, i, pl.ds(16 * u, 16)] = xp * nrm
                    return fcarry
                lax.fori_loop(0, RC, _frow, 0)

                pltpu.sync_copy(
                    fin.at[0, pl.ds(RC, RC)],
                    out.at[pl.ds(rbase, RC), pl.ds(cbase, COLS)])
                pltpu.sync_copy(fin.at[0, pl.ds(0, RC)],
                                s_tab.at[pl.ds(boff + rbase, RC)])
        return carry
    lax.fori_loop(0, K, _hop, 0)


_gpr = pl.kernel(
    _gpr_body,
    out_type=jax.ShapeDtypeStruct((NP_, D), jnp.float32),
    mesh=plsc.VectorSubcoreMesh(core_axis_name="c", subcore_axis_name="s"),
    compiler_params=pltpu.CompilerParams(use_tc_tiling_on_sc=False),
    scratch_types=[
        pltpu.HBM((4 * NP_, COLS), jnp.float32),
        pltpu.VMEM_SHARED((NP_, COLS), jnp.float32),
        pltpu.VMEM((NB, BB), jnp.int32),
        pltpu.VMEM((NB, BB), jnp.int32),
        pltpu.VMEM((RING, BB, COLS), jnp.float32),
        pltpu.VMEM((2, 2 * RC, COLS), jnp.float32),
        pltpu.VMEM((RC, COLS), jnp.float32),
        pltpu.VMEM((NPT, 16), jnp.float32),
        pltpu.VMEM((32,), jnp.float32),
        pltpu.SemaphoreType.DMA,
        pltpu.SemaphoreType.DMA,
        pltpu.SemaphoreType.DMA,
        pltpu.SemaphoreType.DMA,
    ],
)


def kernel(feats, edge_index, temp):
    src = edge_index[0].reshape(NSUB, E // NSUB)
    dst = edge_index[1].reshape(NSUB, E // NSUB)
    src = jnp.pad(src, ((0, 0), (0, EPT - E // NSUB))).reshape(NSUB, NB, BB)
    dst = jnp.pad(dst, ((0, 0), (0, EPT - E // NSUB)),
                  constant_values=NP_ - 1).reshape(NSUB, NB, BB)
    src2 = jnp.stack([src, src + 2 * NP_])
    temp32 = jnp.zeros((32,), jnp.float32).at[: K + 1].set(temp)
    feats_p = jnp.zeros((NP_, D), jnp.float32).at[:N].set(feats)
    return _gpr(feats_p, src2, dst, temp32)[:N]


# speedup vs baseline: 1.3799x; 1.3799x over previous
import jax
import jax.numpy as jnp
from jax import lax
from jax.experimental import pallas as pl
from jax.experimental.pallas import tpu as pltpu
from jax.experimental.pallas import tpu_sc as plsc

N = 10000
NP_ = 10240
E = 160000
D = 256
K = 10
COLS = 64
NSUB = 16
EPT = 10240
BB = 128
NB = EPT // BB
NPT = NP_ // NSUB
RC = 64
NRC = NPT // RC
RV = COLS // 16
RING = 4


def _rsqrt16(d):
    y = jnp.full((16,), 64.0, jnp.float32)
    for _ in range(12):
        y = jnp.float32(0.5) * (y + d / y)
    v = jnp.float32(1.0) / y
    return jnp.where(d == jnp.float32(0.0), jnp.float32(jnp.inf), v)


def _gpr_body(feats, src2, dst_rs, temp32, out,
              s_tab, y_acc,
              src_v, dst_v, rows, fin, zbuf, dchunk, temp_v,
              gsem, ssem, isem, osem):
    c = lax.axis_index("c")
    sid = lax.axis_index("s")
    nbase = sid * NPT

    zero = jnp.zeros((16,), jnp.float32)
    one = jnp.ones((16,), jnp.float32)

    def _zrow(i, carry):
        for u in range(RV):
            zbuf[i, pl.ds(16 * u, 16)] = zero
        return carry
    lax.fori_loop(0, RC, _zrow, 0)

    def _orow(i, carry):
        for u in range(RV):
            rows[0, i, pl.ds(16 * u, 16)] = one
        return carry
    lax.fori_loop(0, BB, _orow, 0)

    pltpu.sync_copy(temp32, temp_v)
    pltpu.sync_copy(src2.at[c, sid], src_v)
    pltpu.sync_copy(dst_rs.at[sid], dst_v)

    def _zero_y():
        for g in range(0, NRC, 5):
            for z in range(g, g + 5):
                pltpu.async_copy(zbuf, y_acc.at[pl.ds(nbase + RC * z, RC)],
                                 osem)
            for z in range(g, g + 5):
                pltpu.make_async_copy(
                    zbuf, y_acc.at[pl.ds(nbase + RC * z, RC)], osem).wait()

    _zero_y()
    plsc.subcore_barrier()
    def _degj(j, carry):
        pltpu.async_copy(rows.at[0], y_acc.at[dst_v.at[j]], ssem, add=True)

        @pl.when(j >= 8)
        def _():
            pltpu.make_async_copy(rows.at[0], y_acc.at[dst_v.at[j]],
                                  ssem).wait()
        return carry
    lax.fori_loop(0, NB, _degj, 0)
    for _ in range(8):
        pltpu.make_async_copy(rows.at[0], y_acc.at[dst_v.at[0]], ssem).wait()
    plsc.subcore_barrier()

    for q in range(NRC):
        pltpu.sync_copy(y_acc.at[pl.ds(nbase + RC * q, RC)],
                        fin.at[0, pl.ds(0, RC)])
        def _nrm(i, carry):
            dchunk[RC * q + i] = _rsqrt16(fin[0, i, pl.ds(0, 16)])
            return carry
        lax.fori_loop(0, RC, _nrm, 0)
    _zero_y()

    t0 = temp_v[pl.ds(0, 16)][0]
    for h in range(2):
        boff = (2 * c + h) * NP_
        cbase = c * 128 + h * COLS
        pltpu.async_copy(feats.at[pl.ds(nbase, RC), pl.ds(cbase, COLS)],
                         fin.at[0, pl.ds(0, RC)], isem)

        @pl.loop(0, NRC, step=2)
        def _init(q0):
            for b in range(2):
                q = q0 + b
                rbase = nbase + RC * q
                pltpu.make_async_copy(
                    feats.at[pl.ds(rbase, RC), pl.ds(cbase, COLS)],
                    fin.at[b, pl.ds(0, RC)], isem).wait()

                def _irow(i, carry):
                    nrm = dchunk[RC * q + i]
                    for u in range(RV):
                        x = fin[b, i, pl.ds(16 * u, 16)]
                        fin[b, RC + i, pl.ds(16 * u, 16)] = x * nrm
                        fin[b, i, pl.ds(16 * u, 16)] = x * t0
                    return carry
                lax.fori_loop(0, RC, _irow, 0)
                pltpu.async_copy(fin.at[b, pl.ds(RC, RC)],
                                 s_tab.at[pl.ds(boff + rbase, RC)], osem)
                pltpu.async_copy(
                    fin.at[b, pl.ds(0, RC)],
                    out.at[pl.ds(rbase, RC), pl.ds(cbase, COLS)], osem)

                @pl.when(q >= 1)
                def _():
                    rb1 = nbase + RC * (q - 1)
                    pltpu.make_async_copy(
                        fin.at[1 - b, pl.ds(RC, RC)],
                        s_tab.at[pl.ds(boff + rb1, RC)], osem).wait()
                    pltpu.make_async_copy(
                        fin.at[1 - b, pl.ds(0, RC)],
                        out.at[pl.ds(rb1, RC), pl.ds(cbase, COLS)],
                        osem).wait()

                @pl.when(q + 1 < NRC)
                def _():
                    rb2 = nbase + RC * (q + 1)
                    pltpu.async_copy(
                        feats.at[pl.ds(rb2, RC), pl.ds(cbase, COLS)],
                        fin.at[1 - b, pl.ds(0, RC)], isem)

        rbl = nbase + RC * (NRC - 1)
        pltpu.make_async_copy(fin.at[1, pl.ds(RC, RC)],
                              s_tab.at[pl.ds(boff + rbl, RC)], osem).wait()
        pltpu.make_async_copy(fin.at[1, pl.ds(0, RC)],
                              out.at[pl.ds(rbl, RC), pl.ds(cbase, COLS)],
                              osem).wait()

    def _hop(k, carry):
        gamma = temp_v[pl.ds(k + 1, 16)][0]
        for h in range(2):
            boff = (2 * c + h) * NP_
            plsc.subcore_barrier()
            pltpu.async_copy(s_tab.at[src_v.at[0]], rows.at[0], gsem)
            pltpu.async_copy(s_tab.at[src_v.at[1]], rows.at[1], gsem)

            @pl.loop(0, NB, step=RING)
            def _edge(j0):
                for b in range(RING):
                    jj = j0 + b
                    pltpu.make_async_copy(s_tab.at[src_v.at[jj]],
                                          rows.at[b], gsem).wait()

                    @pl.when(jj >= 2)
                    def _():
                        pltpu.make_async_copy(
                            rows.at[(b + 2) % RING],
                            y_acc.at[dst_v.at[jj]], ssem).wait()

                    @pl.when(jj + 2 < NB)
                    def _():
                        pltpu.async_copy(s_tab.at[src_v.at[jj + 2]],
                                         rows.at[(b + 2) % RING], gsem)

                    pltpu.async_copy(rows.at[b], y_acc.at[dst_v.at[jj]],
                                     ssem, add=True)

            pltpu.make_async_copy(rows.at[2], y_acc.at[dst_v.at[0]],
                                  ssem).wait()
            pltpu.make_async_copy(rows.at[3], y_acc.at[dst_v.at[0]],
                                  ssem).wait()

            step = NP_ if h == 0 else -NP_
            def _tog(j, carry2):
                for u in range(BB // 16):
                    src_v[j, pl.ds(16 * u, 16)] = (
                        src_v[j, pl.ds(16 * u, 16)] + jnp.int32(step))
                return carry2
            lax.fori_loop(0, NB, _tog, 0)

            plsc.subcore_barrier()
            cbase = c * 128 + h * COLS

            @pl.loop(0, NRC)
            def _fint(q):
                rbase = nbase + RC * q
                pltpu.sync_copy(y_acc.at[pl.ds(rbase, RC)],
                                fin.at[0, pl.ds(0, RC)])
                pltpu.sync_copy(zbuf, y_acc.at[pl.ds(rbase, RC)])
                pltpu.sync_copy(
                    out.at[pl.ds(rbase, RC), pl.ds(cbase, COLS)],
                    fin.at[0, pl.ds(RC, RC)])

                def _frow(i, fcarry):
                    nrm = dchunk[RC * q + i]
                    for u in range(RV):
                        y = fin[0, i, pl.ds(16 * u, 16)]
                        xp = y * nrm
                        fin[0, RC + i, pl.ds(16 * u, 16)] = (
                            fin[0, RC + i, pl.ds(16 * u, 16)]
                            + gamma * xp)
                        fin[0, i, pl.ds(16 * u, 16)] = xp * nrm
                    return fcarry
                lax.fori_loop(0, RC, _frow, 0)

                pltpu.sync_copy(
                    fin.at[0, pl.ds(RC, RC)],
                    out.at[pl.ds(rbase, RC), pl.ds(cbase, COLS)])
                pltpu.sync_copy(fin.at[0, pl.ds(0, RC)],
                                s_tab.at[pl.ds(boff + rbase, RC)])
        return carry
    lax.fori_loop(0, K, _hop, 0)


_gpr = pl.kernel(
    _gpr_body,
    out_type=jax.ShapeDtypeStruct((NP_, D), jnp.float32),
    mesh=plsc.VectorSubcoreMesh(core_axis_name="c", subcore_axis_name="s"),
    compiler_params=pltpu.CompilerParams(use_tc_tiling_on_sc=False),
    scratch_types=[
        pltpu.HBM((4 * NP_, COLS), jnp.float32),
        pltpu.VMEM_SHARED((NP_, COLS), jnp.float32),
        pltpu.VMEM((NB, BB), jnp.int32),
        pltpu.VMEM((NB, BB), jnp.int32),
        pltpu.VMEM((RING, BB, COLS), jnp.float32),
        pltpu.VMEM((2, 2 * RC, COLS), jnp.float32),
        pltpu.VMEM((RC, COLS), jnp.float32),
        pltpu.VMEM((NPT, 16), jnp.float32),
        pltpu.VMEM((32,), jnp.float32),
        pltpu.SemaphoreType.DMA,
        pltpu.SemaphoreType.DMA,
        pltpu.SemaphoreType.DMA,
        pltpu.SemaphoreType.DMA,
    ],
)


def kernel(feats, edge_index, temp):
    src = edge_index[0].reshape(NSUB, E // NSUB)
    dst = edge_index[1].reshape(NSUB, E // NSUB)
    npad = EPT - E // NSUB
    src = jnp.pad(src, ((0, 0), (0, npad))).reshape(NSUB, NB, BB)
    pad_dst = (N + (jnp.arange(NSUB, dtype=jnp.int32)[:, None] * 15
                    + jnp.arange(npad, dtype=jnp.int32)[None, :])
               % (NP_ - N))
    dst = jnp.concatenate([dst, pad_dst], axis=1).reshape(NSUB, NB, BB)
    src2 = jnp.stack([src, src + 2 * NP_])
    temp32 = jnp.zeros((32,), jnp.float32).at[: K + 1].set(temp)
    feats_p = jnp.zeros((NP_, D), jnp.float32).at[:N].set(feats)
    return _gpr(feats_p, src2, dst, temp32)[:N]


# speedup vs baseline: 2.0611x; 1.4937x over previous
import jax
import jax.numpy as jnp
from jax import lax
from jax.experimental import pallas as pl
from jax.experimental.pallas import tpu as pltpu
from jax.experimental.pallas import tpu_sc as plsc

N = 10000
NP_ = 10240
E = 160000
D = 256
K = 10
COLS = 64
NSUB = 16
EPT = E // NSUB
BB = 125
NB = EPT // BB
NPT = NP_ // NSUB
RC = 128
NRC = NPT // RC
ZR = 64
RV = COLS // 16


def _rsqrt16(d):
    y = jnp.full((16,), 64.0, jnp.float32)
    for _ in range(12):
        y = jnp.float32(0.5) * (y + d / y)
    v = jnp.float32(1.0) / y
    return jnp.where(d == jnp.float32(0.0), jnp.float32(jnp.inf), v)


def _gpr_body(feats, src4, dst_rs, temp32, out,
              s_tab, y_acc, deg16,
              src_v, dst_v, rows, fin, zbuf, ones_v, dchunk, temp_v,
              gsem, ssem):
    c = lax.axis_index("c")
    sid = lax.axis_index("s")
    nbase = sid * NPT

    zero = jnp.zeros((16,), jnp.float32)
    one = jnp.ones((16,), jnp.float32)

    def _zrow(i, carry):
        for u in range(RV):
            zbuf[i, pl.ds(16 * u, 16)] = zero
        return carry
    lax.fori_loop(0, ZR, _zrow, 0)

    def _orow(i, carry):
        ones_v[i] = one
        return carry
    lax.fori_loop(0, BB, _orow, 0)

    def _drow(i, carry):
        dchunk[i] = zero
        return carry
    lax.fori_loop(0, NPT, _drow, 0)

    pltpu.sync_copy(temp32, temp_v)
    pltpu.sync_copy(src4.at[c, sid], src_v)
    pltpu.sync_copy(dst_rs.at[sid], dst_v)

    pltpu.sync_copy(dchunk, deg16.at[pl.ds(nbase, NPT)])
    plsc.subcore_barrier()
    def _degj(j, carry):
        pltpu.async_copy(ones_v, deg16.at[dst_v.at[j]], ssem, add=True)

        @pl.when(j >= 8)
        def _():
            pltpu.make_async_copy(ones_v, deg16.at[dst_v.at[j]],
                                  ssem).wait()
        return carry
    lax.fori_loop(0, NB, _degj, 0)
    for _ in range(8):
        pltpu.make_async_copy(ones_v, deg16.at[dst_v.at[0]], ssem).wait()
    plsc.subcore_barrier()
    pltpu.sync_copy(deg16.at[pl.ds(nbase, NPT)], dchunk)

    def _nrm(i, carry):
        dchunk[i] = _rsqrt16(dchunk[i])
        return carry
    lax.fori_loop(0, NPT, _nrm, 0)

    t0 = temp_v[pl.ds(0, 16)][0]
    for h in range(2):
        boff = (2 * c + h) * NP_
        for p in range(NRC):
            rbase = nbase + RC * p
            pltpu.sync_copy(
                feats.at[pl.ds(rbase, RC), pl.ds(c * 128 + h * COLS, COLS)],
                fin.at[pl.ds(0, RC)])
            def _irow(i, carry):
                nrm = dchunk[RC * p + i]
                for u in range(RV):
                    x = fin[i, pl.ds(16 * u, 16)]
                    fin[RC + i, pl.ds(16 * u, 16)] = x * nrm
                    fin[i, pl.ds(16 * u, 16)] = x * t0
                return carry
            lax.fori_loop(0, RC, _irow, 0)
            pltpu.sync_copy(fin.at[pl.ds(RC, RC)],
                            s_tab.at[pl.ds(boff + rbase, RC)])
            pltpu.sync_copy(
                fin.at[pl.ds(0, RC)],
                out.at[pl.ds(rbase, RC), pl.ds(c * 128 + h * COLS, COLS)])
    for z in range(NPT // ZR):
        pltpu.sync_copy(zbuf, y_acc.at[pl.ds(nbase + ZR * z, ZR)])

    def _hop(k, carry):
        gamma = temp_v[pl.ds(k + 1, 16)][0]
        for h in range(2):
            boff = (2 * c + h) * NP_
            plsc.subcore_barrier()
            pltpu.async_copy(s_tab.at[src_v.at[h, 0]], rows.at[0], gsem)

            @pl.loop(0, NB, step=2)
            def _edge(j):
                for b in range(2):
                    jj = j + b
                    pltpu.make_async_copy(
                        s_tab.at[src_v.at[h, jj]], rows.at[b], gsem).wait()

                    @pl.when(jj >= 1)
                    def _():
                        pltpu.make_async_copy(
                            rows.at[1 - b], y_acc.at[dst_v.at[jj]],
                            ssem).wait()

                    @pl.when(jj + 1 < NB)
                    def _():
                        pltpu.async_copy(
                            s_tab.at[src_v.at[h, jj + 1]], rows.at[1 - b],
                            gsem)

                    pltpu.async_copy(
                        rows.at[b], y_acc.at[dst_v.at[jj]], ssem, add=True)

            pltpu.make_async_copy(
                rows.at[1], y_acc.at[dst_v.at[0]], ssem).wait()
            plsc.subcore_barrier()
            for p in range(NRC):
                rbase = nbase + RC * p
                pltpu.sync_copy(y_acc.at[pl.ds(rbase, RC)],
                                fin.at[pl.ds(0, RC)])
                for z in range(RC // ZR):
                    pltpu.sync_copy(
                        zbuf, y_acc.at[pl.ds(rbase + ZR * z, ZR)])
                pltpu.sync_copy(
                    out.at[pl.ds(rbase, RC),
                           pl.ds(c * 128 + h * COLS, COLS)],
                    fin.at[pl.ds(RC, RC)])
                def _frow(i, fcarry):
                    nrm = dchunk[RC * p + i]
                    for u in range(RV):
                        y = fin[i, pl.ds(16 * u, 16)]
                        xp = y * nrm
                        fin[RC + i, pl.ds(16 * u, 16)] = (
                            fin[RC + i, pl.ds(16 * u, 16)] + gamma * xp)
                        fin[i, pl.ds(16 * u, 16)] = xp * nrm
                    return fcarry
                lax.fori_loop(0, RC, _frow, 0)
                pltpu.sync_copy(
                    fin.at[pl.ds(RC, RC)],
                    out.at[pl.ds(rbase, RC),
                           pl.ds(c * 128 + h * COLS, COLS)])
                pltpu.sync_copy(fin.at[pl.ds(0, RC)],
                                s_tab.at[pl.ds(boff + rbase, RC)])
        return carry
    lax.fori_loop(0, K, _hop, 0)


_gpr = pl.kernel(
    _gpr_body,
    out_type=jax.ShapeDtypeStruct((NP_, D), jnp.float32),
    mesh=plsc.VectorSubcoreMesh(core_axis_name="c", subcore_axis_name="s"),
    compiler_params=pltpu.CompilerParams(use_tc_tiling_on_sc=False),
    scratch_types=[
        pltpu.HBM((4 * NP_, COLS), jnp.float32),
        pltpu.VMEM_SHARED((NP_, COLS), jnp.float32),
        pltpu.VMEM_SHARED((NP_, 16), jnp.float32),
        pltpu.VMEM((2, NB, BB), jnp.int32),
        pltpu.VMEM((NB, BB), jnp.int32),
        pltpu.VMEM((2, BB, COLS), jnp.float32),
        pltpu.VMEM((2 * RC, COLS), jnp.float32),
        pltpu.VMEM((ZR, COLS), jnp.float32),
        pltpu.VMEM((BB, 16), jnp.float32),
        pltpu.VMEM((NPT, 16), jnp.float32),
        pltpu.VMEM((32,), jnp.float32),
        pltpu.SemaphoreType.DMA,
        pltpu.SemaphoreType.DMA,
    ],
)


def kernel(feats, edge_index, temp):
    src = edge_index[0].reshape(NSUB, NB, BB)
    dst = edge_index[1].reshape(NSUB, NB, BB)
    src4 = jnp.stack([jnp.stack([src, src + NP_]),
                      jnp.stack([src + 2 * NP_, src + 3 * NP_])])
    src4 = src4.transpose(0, 2, 1, 3, 4)
    temp32 = jnp.zeros((32,), jnp.float32).at[: K + 1].set(temp)
    feats_p = jnp.zeros((NP_, D), jnp.float32).at[:N].set(feats)
    return _gpr(feats_p, src4, dst, temp32)[:N]
